# baseline (speedup 1.0000x reference)
import functools

import jax
import jax.numpy as jnp
from jax import lax
from jax.experimental import pallas as pl
from jax.experimental.pallas import tpu as pltpu
from jax.experimental.pallas import tpu_sc as plsc

NC = 2
NS = 16
NW = NC * NS
CHUNK = 400
LANES = 16


def _round_up(n, m):
    return (n + m - 1) // m * m


@functools.lru_cache(maxsize=None)
def _build(b_node_pad, b_edge_pad, vocab, dim):
    n_w_n = b_node_pad // NW
    n_w_e = b_edge_pad // NW
    mesh = plsc.VectorSubcoreMesh(
        core_axis_name="c", subcore_axis_name="s", num_cores=NC, num_subcores=NS
    )

    @functools.partial(
        pl.kernel,
        mesh=mesh,
        compiler_params=pltpu.CompilerParams(needs_layout_passes=False),
        out_type=[
            jax.ShapeDtypeStruct((b_node_pad * dim,), jnp.float32),
            jax.ShapeDtypeStruct((b_edge_pad * dim,), jnp.float32),
        ],
        scratch_types=[
            pltpu.VMEM((CHUNK,), jnp.int32),
            pltpu.VMEM((CHUNK,), jnp.int32),
            pltpu.VMEM((CHUNK,), jnp.int32),
            pltpu.VMEM((CHUNK,), jnp.int32),
            pltpu.VMEM((CHUNK,), jnp.int32),
            pltpu.VMEM((CHUNK,), jnp.int32),
            pltpu.VMEM((2, CHUNK, 128), jnp.float32),
            pltpu.VMEM((CHUNK * dim,), jnp.float32),
            pltpu.SemaphoreType.DMA,
            pltpu.SemaphoreType.DMA,
            pltpu.SemaphoreType.DMA,
            pltpu.SemaphoreType.DMA,
            pltpu.SemaphoreType.DMA,
        ],
    )
    def emb_kernel(x_hbm, e_hbm, ntab4, etab4, out_n1, out_e1,
                   ixa, ixb, ix4a, ix4b, qa, qb, rows_v, stage_v,
                   i0, i1, g0, g1, wsem):
        wid = lax.axis_index("s") * NC + lax.axis_index("c")
        sem_i = (i0, i1)
        sem_g = (g0, g1)
        idx_v = (ixa, ixb)
        idx4_v = (ix4a, ix4b)
        q_v = (qa, qb)

        def phase(idx_hbm, tab_hbm, out_hbm, n_w, first):
            n = n_w // CHUNK
            base = wid * n_w
            obase = base * dim

            def ix_start(i, b):
                off = pl.multiple_of(base + i * CHUNK, 8)
                pltpu.async_copy(
                    idx_hbm.at[pl.ds(off, CHUNK)], idx_v[b], sem_i[b]
                )

            def ix_wait(b):
                pltpu.make_async_copy(
                    idx_hbm.at[pl.ds(base, CHUNK)], idx_v[b], sem_i[b]
                ).wait()

            def prep(b):
                for t in range(CHUNK // LANES):
                    v = idx_v[b][pl.ds(t * LANES, LANES)]
                    idx4_v[b][pl.ds(t * LANES, LANES)] = v >> 2
                    q_v[b][pl.ds(t * LANES, LANES)] = v & 3

            def g_start(b):
                pltpu.async_copy(tab_hbm.at[idx4_v[b]], rows_v.at[b], sem_g[b])

            def g_wait(b):
                pltpu.make_async_copy(
                    tab_hbm.at[idx4_v[b]], rows_v.at[b], sem_g[b]
                ).wait()

            def w_start(i):
                off = pl.multiple_of(obase + i * CHUNK * dim, 8)
                pltpu.async_copy(
                    stage_v, out_hbm.at[pl.ds(off, CHUNK * dim)], wsem
                )

            def w_wait():
                pltpu.make_async_copy(
                    stage_v, out_hbm.at[pl.ds(obase, CHUNK * dim)], wsem
                ).wait()

            def repack(b):
                iota = lax.iota(jnp.int32, LANES)

                @pl.loop(0, CHUNK // LANES)
                def _(t):
                    rvec = t * LANES + iota
                    q16 = q_v[b][pl.ds(t * LANES, LANES)]
                    cbase = q16 * dim
                    sbase = rvec * dim
                    for d in range(dim):
                        val = plsc.load_gather(rows_v.at[b], [rvec, cbase + d])
                        plsc.store_scatter(stage_v, [sbase + d], val)

            def process(ib, b):
                bo = 1 - b

                @pl.when(ib + 1 < n)
                def _():
                    ix_wait(bo)
                    prep(bo)
                    g_start(bo)
                g_wait(b)

                @pl.when(ib + 2 < n)
                def _():
                    ix_start(ib + 2, b)
                if first:
                    @pl.when(ib >= 1)
                    def _():
                        w_wait()
                else:
                    w_wait()
                repack(b)
                w_start(ib)

            ix_start(0, 0)
            ix_start(1, 1)
            ix_wait(0)
            prep(0)
            g_start(0)

            @pl.loop(0, n)
            def _(i):
                for bp in range(2):
                    @pl.when(lax.rem(i, 2) == bp)
                    def _():
                        process(i, bp)

        phase(e_hbm, etab4, out_e1, n_w_e, True)
        phase(x_hbm, ntab4, out_n1, n_w_n, False)
        pltpu.make_async_copy(
            stage_v, out_n1.at[pl.ds(0, CHUNK * dim)], wsem
        ).wait()

    return emb_kernel


def kernel(x, edge_attr, node_table, edge_table):
    b_n = x.shape[0]
    b_e = edge_attr.shape[0]
    vocab, dim = node_table.shape
    b_n_pad = _round_up(b_n, NW * CHUNK)
    b_e_pad = _round_up(b_e, NW * CHUNK)
    x_i = jnp.pad(x.astype(jnp.int32), (0, b_n_pad - b_n))
    e_i = jnp.pad(edge_attr.astype(jnp.int32), (0, b_e_pad - b_e))
    ntab4 = node_table.reshape(vocab * dim // 128, 128)
    etab4 = edge_table.reshape(vocab * dim // 128, 128)
    out_n1, out_e1 = _build(b_n_pad, b_e_pad, vocab, dim)(x_i, e_i, ntab4, etab4)
    out_n = out_n1.reshape(b_n_pad, dim)[:b_n]
    out_e = out_e1.reshape(b_e_pad, dim)[:b_e]
    return (out_n, out_e)


# speedup vs baseline: 1.3566x; 1.3566x over previous
import functools

import jax
import jax.numpy as jnp
from jax import lax
from jax.experimental import pallas as pl
from jax.experimental.pallas import tpu as pltpu
from jax.experimental.pallas import tpu_sc as plsc

NC = 2
NS = 16
NW = NC * NS
CHUNK = 384
LANES = 16


def _round_up(n, m):
    return (n + m - 1) // m * m


@functools.lru_cache(maxsize=None)
def _build(b_node_pad, b_edge_pad, vocab, dim):
    n_w_n = b_node_pad // NW
    n_w_e = b_edge_pad // NW
    mesh = plsc.VectorSubcoreMesh(
        core_axis_name="c", subcore_axis_name="s", num_cores=NC, num_subcores=NS
    )

    @functools.partial(
        pl.kernel,
        mesh=mesh,
        compiler_params=pltpu.CompilerParams(
            use_tc_tiling_on_sc=True, needs_layout_passes=False
        ),
        out_type=[
            jax.ShapeDtypeStruct((dim, b_node_pad), jnp.float32),
            jax.ShapeDtypeStruct((dim, b_edge_pad), jnp.float32),
        ],
        scratch_types=[
            pltpu.VMEM((CHUNK,), jnp.int32),
            pltpu.VMEM((CHUNK,), jnp.int32),
            pltpu.VMEM((CHUNK,), jnp.int32),
            pltpu.VMEM((CHUNK,), jnp.int32),
            pltpu.VMEM((CHUNK,), jnp.int32),
            pltpu.VMEM((CHUNK,), jnp.int32),
            pltpu.VMEM((2, CHUNK, 128), jnp.float32),
            pltpu.VMEM((dim, CHUNK), jnp.float32),
            pltpu.SemaphoreType.DMA,
            pltpu.SemaphoreType.DMA,
            pltpu.SemaphoreType.DMA,
            pltpu.SemaphoreType.DMA,
            pltpu.SemaphoreType.DMA,
        ],
    )
    def emb_kernel(x_hbm, e_hbm, ntab4, etab4, out_n1, out_e1,
                   ixa, ixb, ix4a, ix4b, qa, qb, rows_v, stage_v,
                   i0, i1, g0, g1, wsem):
        wid = lax.axis_index("s") * NC + lax.axis_index("c")
        sem_i = (i0, i1)
        sem_g = (g0, g1)
        idx_v = (ixa, ixb)
        idx4_v = (ix4a, ix4b)
        q_v = (qa, qb)

        def phase(idx_hbm, tab_hbm, out_hbm, n_w, first):
            n = n_w // CHUNK
            base = wid * n_w
            obase = base * dim

            def ix_start(i, b):
                off = pl.multiple_of(base + i * CHUNK, 8)
                pltpu.async_copy(
                    idx_hbm.at[pl.ds(off, CHUNK)], idx_v[b], sem_i[b]
                )

            def ix_wait(b):
                pltpu.make_async_copy(
                    idx_hbm.at[pl.ds(base, CHUNK)], idx_v[b], sem_i[b]
                ).wait()

            def prep(b):
                for t in range(CHUNK // LANES):
                    v = idx_v[b][pl.ds(t * LANES, LANES)]
                    idx4_v[b][pl.ds(t * LANES, LANES)] = v >> 2
                    q_v[b][pl.ds(t * LANES, LANES)] = v & 3

            def g_start(b):
                pltpu.async_copy(tab_hbm.at[idx4_v[b]], rows_v.at[b], sem_g[b])

            def g_wait(b):
                pltpu.make_async_copy(
                    tab_hbm.at[idx4_v[b]], rows_v.at[b], sem_g[b]
                ).wait()

            def w_start(i):
                off = pl.multiple_of(base + i * CHUNK, 128)
                pltpu.async_copy(
                    stage_v, out_hbm.at[:, pl.ds(off, CHUNK)], wsem
                )

            def w_wait():
                pltpu.make_async_copy(
                    stage_v, out_hbm.at[:, pl.ds(base, CHUNK)], wsem
                ).wait()

            def repack(b):
                iota = lax.iota(jnp.int32, LANES)

                @pl.loop(0, CHUNK // LANES, unroll=3)
                def _(t):
                    rvec = t * LANES + iota
                    q16 = q_v[b][pl.ds(t * LANES, LANES)]
                    cbase = q16 * dim
                    for d in range(dim):
                        val = plsc.load_gather(rows_v.at[b], [rvec, cbase + d])
                        stage_v[d, pl.ds(t * LANES, LANES)] = val

            def process(ib, b):
                bo = 1 - b

                @pl.when(ib + 1 < n)
                def _():
                    ix_wait(bo)
                    prep(bo)
                    g_start(bo)
                g_wait(b)

                @pl.when(ib + 2 < n)
                def _():
                    ix_start(ib + 2, b)
                if first:
                    @pl.when(ib >= 1)
                    def _():
                        w_wait()
                else:
                    w_wait()
                repack(b)
                w_start(ib)

            ix_start(0, 0)
            ix_start(1, 1)
            ix_wait(0)
            prep(0)
            g_start(0)

            @pl.loop(0, n)
            def _(i):
                for bp in range(2):
                    @pl.when(lax.rem(i, 2) == bp)
                    def _():
                        process(i, bp)

        phase(e_hbm, etab4, out_e1, n_w_e, True)
        phase(x_hbm, ntab4, out_n1, n_w_n, False)
        pltpu.make_async_copy(
            stage_v, out_n1.at[:, pl.ds(0, CHUNK)], wsem
        ).wait()

    return emb_kernel


def kernel(x, edge_attr, node_table, edge_table):
    b_n = x.shape[0]
    b_e = edge_attr.shape[0]
    vocab, dim = node_table.shape
    b_n_pad = _round_up(b_n, NW * CHUNK)
    b_e_pad = _round_up(b_e, NW * CHUNK)
    x_i = jnp.pad(x.astype(jnp.int32), (0, b_n_pad - b_n))
    e_i = jnp.pad(edge_attr.astype(jnp.int32), (0, b_e_pad - b_e))
    ntab4 = node_table.reshape(vocab * dim // 128, 128)
    etab4 = edge_table.reshape(vocab * dim // 128, 128)
    out_n1, out_e1 = _build(b_n_pad, b_e_pad, vocab, dim)(x_i, e_i, ntab4, etab4)
    out_n = out_n1.T[:b_n]
    out_e = out_e1.T[:b_e]
    return (out_n, out_e)


# speedup vs baseline: 1.9943x; 1.4700x over previous
import functools

import jax
import jax.numpy as jnp
from jax import lax
from jax.experimental import pallas as pl
from jax.experimental.pallas import tpu as pltpu
from jax.experimental.pallas import tpu_sc as plsc

NC = 2
NS = 16
NW = NC * NS
CHUNK = 2000


def _round_up(n, m):
    return (n + m - 1) // m * m


@functools.lru_cache(maxsize=None)
def _build(b_node_pad, b_edge_pad, dim):
    n_w_n = b_node_pad // NW
    n_w_e = b_edge_pad // NW
    mesh = plsc.VectorSubcoreMesh(
        core_axis_name="c", subcore_axis_name="s", num_cores=NC, num_subcores=NS
    )

    @functools.partial(
        pl.kernel,
        mesh=mesh,
        compiler_params=pltpu.CompilerParams(use_tc_tiling_on_sc=False),
        out_type=[
            jax.ShapeDtypeStruct((b_node_pad, dim), jnp.float32),
            jax.ShapeDtypeStruct((b_edge_pad, dim), jnp.float32),
        ],
        scratch_types=[
            pltpu.VMEM((CHUNK,), jnp.int32),
            pltpu.VMEM((CHUNK, dim), jnp.float32),
            pltpu.SemaphoreType.DMA,
        ],
    )
    def emb_kernel(x_hbm, e_hbm, ntab, etab, out_n, out_e, idx_v, rows_v, sem):
        wid = lax.axis_index("s") * NC + lax.axis_index("c")

        def do_chunk(idx_hbm, tab_hbm, out_hbm, off, size):
            pltpu.sync_copy(idx_hbm.at[pl.ds(off, size)], idx_v.at[pl.ds(0, size)])
            pltpu.async_copy(
                tab_hbm.at[idx_v.at[pl.ds(0, size)]],
                rows_v.at[pl.ds(0, size)],
                sem,
            ).wait()
            pltpu.sync_copy(rows_v.at[pl.ds(0, size)], out_hbm.at[pl.ds(off, size)])

        def phase(idx_hbm, tab_hbm, out_hbm, n_w):
            base = wid * n_w
            k_full = n_w // CHUNK
            rem = n_w % CHUNK
            if k_full:
                @pl.loop(0, k_full)
                def _(i):
                    off = pl.multiple_of(base + i * CHUNK, 8)
                    do_chunk(idx_hbm, tab_hbm, out_hbm, off, CHUNK)
            if rem:
                off = pl.multiple_of(base + k_full * CHUNK, 8)
                do_chunk(idx_hbm, tab_hbm, out_hbm, off, rem)

        phase(e_hbm, etab, out_e, n_w_e)
        phase(x_hbm, ntab, out_n, n_w_n)

    return emb_kernel


def kernel(x, edge_attr, node_table, edge_table):
    b_n = x.shape[0]
    b_e = edge_attr.shape[0]
    dim = node_table.shape[1]
    b_n_pad = _round_up(b_n, NW * 8)
    b_e_pad = _round_up(b_e, NW * 8)
    x_i = jnp.pad(x.astype(jnp.int32), (0, b_n_pad - b_n))
    e_i = jnp.pad(edge_attr.astype(jnp.int32), (0, b_e_pad - b_e))
    out_n, out_e = _build(b_n_pad, b_e_pad, dim)(x_i, e_i, node_table, edge_table)
    return (out_n[:b_n], out_e[:b_e])


# speedup vs baseline: 2.0103x; 1.0080x over previous
import functools

import jax
import jax.numpy as jnp
from jax import lax
from jax.experimental import pallas as pl
from jax.experimental.pallas import tpu as pltpu
from jax.experimental.pallas import tpu_sc as plsc

NC = 2
NS = 16
NW = NC * NS
CHUNK = 3000


def _round_up(n, m):
    return (n + m - 1) // m * m


@functools.lru_cache(maxsize=None)
def _build(b_node_pad, b_edge_pad, dim):
    n_w_n = b_node_pad // NW
    n_w_e = b_edge_pad // NW
    mesh = plsc.VectorSubcoreMesh(
        core_axis_name="c", subcore_axis_name="s", num_cores=NC, num_subcores=NS
    )

    @functools.partial(
        pl.kernel,
        mesh=mesh,
        compiler_params=pltpu.CompilerParams(use_tc_tiling_on_sc=False),
        out_type=[
            jax.ShapeDtypeStruct((b_node_pad, dim), jnp.float32),
            jax.ShapeDtypeStruct((b_edge_pad, dim), jnp.float32),
        ],
        scratch_types=[
            pltpu.VMEM((CHUNK,), jnp.int32),
            pltpu.VMEM((CHUNK, dim), jnp.float32),
            pltpu.SemaphoreType.DMA,
        ],
    )
    def emb_kernel(x_hbm, e_hbm, ntab, etab, out_n, out_e, idx_v, rows_v, sem):
        wid = lax.axis_index("s") * NC + lax.axis_index("c")

        def do_chunk(idx_hbm, tab_hbm, out_hbm, off, size):
            pltpu.sync_copy(idx_hbm.at[pl.ds(off, size)], idx_v.at[pl.ds(0, size)])
            pltpu.async_copy(
                tab_hbm.at[idx_v.at[pl.ds(0, size)]],
                rows_v.at[pl.ds(0, size)],
                sem,
            ).wait()
            pltpu.sync_copy(rows_v.at[pl.ds(0, size)], out_hbm.at[pl.ds(off, size)])

        def phase(idx_hbm, tab_hbm, out_hbm, n_w):
            base = wid * n_w
            k_full = n_w // CHUNK
            rem = n_w % CHUNK
            if k_full:
                @pl.loop(0, k_full)
                def _(i):
                    off = pl.multiple_of(base + i * CHUNK, 8)
                    do_chunk(idx_hbm, tab_hbm, out_hbm, off, CHUNK)
            if rem:
                off = pl.multiple_of(base + k_full * CHUNK, 8)
                do_chunk(idx_hbm, tab_hbm, out_hbm, off, rem)

        phase(e_hbm, etab, out_e, n_w_e)
        phase(x_hbm, ntab, out_n, n_w_n)

    return emb_kernel


def kernel(x, edge_attr, node_table, edge_table):
    b_n = x.shape[0]
    b_e = edge_attr.shape[0]
    dim = node_table.shape[1]
    b_n_pad = _round_up(b_n, NW * 8)
    b_e_pad = _round_up(b_e, NW * 8)
    x_i = jnp.pad(x.astype(jnp.int32), (0, b_n_pad - b_n))
    e_i = jnp.pad(edge_attr.astype(jnp.int32), (0, b_e_pad - b_e))
    out_n, out_e = _build(b_n_pad, b_e_pad, dim)(x_i, e_i, node_table, edge_table)
    return (out_n[:b_n], out_e[:b_e])


# speedup vs baseline: 2.0243x; 1.0069x over previous
import functools

import jax
import jax.numpy as jnp
from jax import lax
from jax.experimental import pallas as pl
from jax.experimental.pallas import tpu as pltpu
from jax.experimental.pallas import tpu_sc as plsc

NC = 2
NS = 16
NW = NC * NS
CHUNK = 3000


def _round_up(n, m):
    return (n + m - 1) // m * m


@functools.lru_cache(maxsize=None)
def _build(b_pad, dim):
    n_w = b_pad // NW
    mesh = plsc.VectorSubcoreMesh(
        core_axis_name="c", subcore_axis_name="s", num_cores=NC, num_subcores=NS
    )

    @functools.partial(
        pl.kernel,
        mesh=mesh,
        compiler_params=pltpu.CompilerParams(use_tc_tiling_on_sc=False),
        out_type=jax.ShapeDtypeStruct((b_pad, dim), jnp.float32),
        scratch_types=[
            pltpu.VMEM((CHUNK,), jnp.int32),
            pltpu.VMEM((CHUNK, dim), jnp.float32),
            pltpu.SemaphoreType.DMA,
        ],
    )
    def emb_kernel(idx_hbm, tab_hbm, out_hbm, idx_v, rows_v, sem):
        wid = lax.axis_index("s") * NC + lax.axis_index("c")

        def do_chunk(off, size):
            pltpu.sync_copy(idx_hbm.at[pl.ds(off, size)], idx_v.at[pl.ds(0, size)])
            pltpu.async_copy(
                tab_hbm.at[idx_v.at[pl.ds(0, size)]],
                rows_v.at[pl.ds(0, size)],
                sem,
            ).wait()
            pltpu.sync_copy(rows_v.at[pl.ds(0, size)], out_hbm.at[pl.ds(off, size)])

        base = wid * n_w
        k_full = n_w // CHUNK
        rem = n_w % CHUNK
        if k_full:
            @pl.loop(0, k_full)
            def _(i):
                off = pl.multiple_of(base + i * CHUNK, 8)
                do_chunk(off, CHUNK)
        if rem:
            off = pl.multiple_of(base + k_full * CHUNK, 8)
            do_chunk(off, rem)

    return emb_kernel


def _lookup(idx, table):
    b = idx.shape[0]
    dim = table.shape[1]
    b_pad = _round_up(b, NW * 8)
    idx_i = jnp.pad(idx.astype(jnp.int32), (0, b_pad - b))
    out = _build(b_pad, dim)(idx_i, table)
    return out[:b]


def kernel(x, edge_attr, node_table, edge_table):
    return (_lookup(x, node_table), _lookup(edge_attr, edge_table))


# speedup vs baseline: 2.0282x; 1.0019x over previous
import functools

import jax
import jax.numpy as jnp
from jax import lax
from jax.experimental import pallas as pl
from jax.experimental.pallas import tpu as pltpu
from jax.experimental.pallas import tpu_sc as plsc

NC = 2
NS = 16
NW = NC * NS
CHUNK = 3500


def _round_up(n, m):
    return (n + m - 1) // m * m


@functools.lru_cache(maxsize=None)
def _build(b_pad, dim):
    n_w = b_pad // NW
    mesh = plsc.VectorSubcoreMesh(
        core_axis_name="c", subcore_axis_name="s", num_cores=NC, num_subcores=NS
    )

    @functools.partial(
        pl.kernel,
        mesh=mesh,
        compiler_params=pltpu.CompilerParams(use_tc_tiling_on_sc=False),
        out_type=jax.ShapeDtypeStruct((b_pad, dim), jnp.float32),
        scratch_types=[
            pltpu.VMEM((CHUNK,), jnp.int32),
            pltpu.VMEM((CHUNK, dim), jnp.float32),
            pltpu.SemaphoreType.DMA,
        ],
    )
    def emb_kernel(idx_hbm, tab_hbm, out_hbm, idx_v, rows_v, sem):
        wid = lax.axis_index("s") * NC + lax.axis_index("c")

        def do_chunk(off, size):
            pltpu.sync_copy(idx_hbm.at[pl.ds(off, size)], idx_v.at[pl.ds(0, size)])
            pltpu.async_copy(
                tab_hbm.at[idx_v.at[pl.ds(0, size)]],
                rows_v.at[pl.ds(0, size)],
                sem,
            ).wait()
            pltpu.sync_copy(rows_v.at[pl.ds(0, size)], out_hbm.at[pl.ds(off, size)])

        base = wid * n_w
        k_full = n_w // CHUNK
        rem = n_w % CHUNK
        if k_full:
            @pl.loop(0, k_full)
            def _(i):
                off = pl.multiple_of(base + i * CHUNK, 8)
                do_chunk(off, CHUNK)
        if rem:
            off = pl.multiple_of(base + k_full * CHUNK, 8)
            do_chunk(off, rem)

    return emb_kernel


def _lookup(idx, table):
    b = idx.shape[0]
    dim = table.shape[1]
    b_pad = _round_up(b, NW * 8)
    idx_i = jnp.pad(idx.astype(jnp.int32), (0, b_pad - b))
    out = _build(b_pad, dim)(idx_i, table)
    return out[:b]


def kernel(x, edge_attr, node_table, edge_table):
    return (_lookup(x, node_table), _lookup(edge_attr, edge_table))
